# baseline (speedup 1.0000x reference)
import functools

import jax
import jax.numpy as jnp
from jax import lax
from jax.experimental import pallas as pl
from jax.experimental.pallas import tpu as pltpu
from jax.experimental.pallas import tpu_sc as plsc

NUM_CORES = 2
NUM_SUBCORES = 16
NW = NUM_CORES * NUM_SUBCORES
LANES = 128


def _tc_pack_table(At, blkv):
    d, vocab = At.shape
    assert blkv % LANES == 0
    grid = -(-vocab // blkv)

    def pack_kernel(at_ref, out_ref):
        out_ref[:, :d] = at_ref[...].T

    return pl.pallas_call(
        pack_kernel,
        grid=(grid,),
        in_specs=[pl.BlockSpec((d, blkv), lambda i: (0, i))],
        out_specs=pl.BlockSpec((blkv, LANES), lambda i: (i, 0)),
        out_shape=jax.ShapeDtypeStruct((vocab, LANES), jnp.float32),
        compiler_params=pltpu.CompilerParams(
            dimension_semantics=("parallel",)),
    )(At)


def _sc_transpose_table(At, chunk):
    d, vocab = At.shape
    nch = -(-vocab // chunk)
    nt = -(-nch // NW)
    assert chunk % 8 == 0 and vocab % chunk == 0

    mesh = plsc.VectorSubcoreMesh(core_axis_name="c", subcore_axis_name="s")

    @functools.partial(
        pl.kernel,
        mesh=mesh,
        out_type=jax.ShapeDtypeStruct((vocab, d), jnp.float32),
        scratch_types=[
            pltpu.VMEM((d, chunk), jnp.float32),
            pltpu.VMEM((chunk, d), jnp.float32),
        ],
        compiler_params=pltpu.CompilerParams(
            use_tc_tiling_on_sc=False, needs_layout_passes=False),
    )
    def transpose_kernel(at_hbm, out_hbm, ibuf, obuf):
        wid = lax.axis_index("s") * NUM_CORES + lax.axis_index("c")
        rows = lax.broadcasted_iota(jnp.int32, (d,), 0)

        @pl.loop(0, nt)
        def _(t):
            c = t * NW + wid

            @pl.when(c < nch)
            def _():
                c0 = c * chunk
                pltpu.sync_copy(at_hbm.at[:, pl.ds(c0, chunk)], ibuf)

                @pl.loop(0, chunk)
                def _(v):
                    col = plsc.load_gather(
                        ibuf, [rows, jnp.zeros((d,), jnp.int32) + v])
                    obuf[v, :] = col

                pltpu.sync_copy(obuf, out_hbm.at[pl.ds(c0, chunk)])

    return transpose_kernel(At)


def _sc_gather(table, idx, chunk):
    n = idx.shape[0]
    d = table.shape[1]
    b_per_w = n // NW
    nch = b_per_w // chunk
    assert b_per_w * NW == n and nch * chunk == b_per_w and chunk % 8 == 0

    mesh = plsc.VectorSubcoreMesh(core_axis_name="c", subcore_axis_name="s")

    @functools.partial(
        pl.kernel,
        mesh=mesh,
        out_type=jax.ShapeDtypeStruct((n, d), jnp.float32),
        scratch_types=[
            pltpu.VMEM((b_per_w,), jnp.int32),
            pltpu.VMEM((chunk, d), jnp.float32),
            pltpu.VMEM((chunk, d), jnp.float32),
            pltpu.SemaphoreType.DMA,
            pltpu.SemaphoreType.DMA,
            pltpu.SemaphoreType.DMA,
            pltpu.SemaphoreType.DMA,
        ],
        compiler_params=pltpu.CompilerParams(use_tc_tiling_on_sc=False),
    )
    def gather_kernel(table_hbm, idx_hbm, out_hbm, idx_v, buf0, buf1,
                      gs0, gs1, os0, os1):
        wid = lax.axis_index("s") * NUM_CORES + lax.axis_index("c")
        base = wid * b_per_w
        pltpu.sync_copy(idx_hbm.at[pl.ds(base, b_per_w)], idx_v)

        bufs = (buf0, buf1)
        gsems = (gs0, gs1)
        osems = (os0, os1)

        def start_gather(c, buf, sem):
            return pltpu.async_copy(
                table_hbm.at[idx_v.at[pl.ds(c * chunk, chunk)]], buf, sem)

        gcp = [start_gather(0, bufs[0], gsems[0]), None]
        ocp = [None, None]
        for c in range(nch):
            cur = c & 1
            nxt = 1 - cur
            if c + 1 < nch:
                if ocp[nxt] is not None:
                    ocp[nxt].wait()
                gcp[nxt] = start_gather(c + 1, bufs[nxt], gsems[nxt])
            gcp[cur].wait()
            ocp[cur] = pltpu.async_copy(
                bufs[cur], out_hbm.at[pl.ds(base + c * chunk, chunk)],
                osems[cur])
        for cp in ocp:
            if cp is not None:
                cp.wait()

    return gather_kernel(table, idx)


def _tc_project_t(emb512, W_pad, bias_col, bblk):
    batch, fl = emb512.shape
    nrow = W_pad.shape[0]
    assert batch % bblk == 0

    def proj_kernel(emb_ref, w_ref, b_ref, out_ref):
        out_ref[...] = (
            lax.dot_general(w_ref[...], emb_ref[...],
                            (((1,), (1,)), ((), ())),
                            preferred_element_type=jnp.float32)
            + b_ref[...]
        )

    return pl.pallas_call(
        proj_kernel,
        grid=(batch // bblk,),
        in_specs=[
            pl.BlockSpec((bblk, fl), lambda i: (i, 0)),
            pl.BlockSpec((nrow, fl), lambda i: (0, 0)),
            pl.BlockSpec((nrow, 1), lambda i: (0, 0)),
        ],
        out_specs=pl.BlockSpec((nrow, bblk), lambda i: (0, i)),
        out_shape=jax.ShapeDtypeStruct((nrow, batch), jnp.float32),
        compiler_params=pltpu.CompilerParams(
            dimension_semantics=("parallel",)),
    )(emb512, W_pad, bias_col)


def kernel(x, A, B_w, B_b):
    batch, fields = x.shape
    vocab, rank = A.shape
    embed = B_w.shape[0]
    fpad = 32

    table = _sc_transpose_table(A.T, chunk=2000)

    idx = jnp.concatenate([x, x[:, :fpad - fields]], axis=1).reshape(
        batch * fpad)
    emb = _sc_gather(table, idx, chunk=2048)
    emb512 = emb.reshape(batch, fpad * rank)

    W_pad = jnp.kron(jnp.eye(fields, fpad, dtype=jnp.float32), B_w)
    bias_col = jnp.tile(B_b, fields).reshape(fields * embed, 1)
    out_t = _tc_project_t(emb512, W_pad, bias_col, bblk=1024)

    return out_t.reshape(fields, embed, batch).transpose(2, 0, 1)


# speedup vs baseline: 1.4474x; 1.4474x over previous
import functools

import jax
import jax.numpy as jnp
from jax import lax
from jax.experimental import pallas as pl
from jax.experimental.pallas import tpu as pltpu
from jax.experimental.pallas import tpu_sc as plsc

NUM_CORES = 2
NUM_SUBCORES = 16
NW = NUM_CORES * NUM_SUBCORES
LANES = 128


def _tc_pack_table(At, blkv):
    d, vocab = At.shape
    assert blkv % LANES == 0
    grid = -(-vocab // blkv)

    def pack_kernel(at_ref, out_ref):
        out_ref[:, :d] = at_ref[...].T.astype(jnp.bfloat16)

    return pl.pallas_call(
        pack_kernel,
        grid=(grid,),
        in_specs=[pl.BlockSpec((d, blkv), lambda i: (0, i))],
        out_specs=pl.BlockSpec((blkv, LANES), lambda i: (i, 0)),
        out_shape=jax.ShapeDtypeStruct((vocab, LANES), jnp.bfloat16),
        compiler_params=pltpu.CompilerParams(
            dimension_semantics=("parallel",)),
    )(At)


def _sc_gather(table, idx, chunk):
    n = idx.shape[0]
    d = table.shape[1]
    b_per_w = n // NW
    nch = b_per_w // chunk
    assert b_per_w * NW == n and nch * chunk == b_per_w and chunk % 8 == 0

    mesh = plsc.VectorSubcoreMesh(core_axis_name="c", subcore_axis_name="s")

    @functools.partial(
        pl.kernel,
        mesh=mesh,
        out_type=jax.ShapeDtypeStruct((n, d), table.dtype),
        scratch_types=[
            pltpu.VMEM((b_per_w,), jnp.int32),
            pltpu.VMEM((chunk, d), table.dtype),
            pltpu.VMEM((chunk, d), table.dtype),
            pltpu.SemaphoreType.DMA,
            pltpu.SemaphoreType.DMA,
            pltpu.SemaphoreType.DMA,
            pltpu.SemaphoreType.DMA,
        ],
        compiler_params=pltpu.CompilerParams(use_tc_tiling_on_sc=False),
    )
    def gather_kernel(table_hbm, idx_hbm, out_hbm, idx_v, buf0, buf1,
                      gs0, gs1, os0, os1):
        wid = lax.axis_index("s") * NUM_CORES + lax.axis_index("c")
        base = wid * b_per_w
        pltpu.sync_copy(idx_hbm.at[pl.ds(base, b_per_w)], idx_v)

        bufs = (buf0, buf1)
        gsems = (gs0, gs1)
        osems = (os0, os1)

        def start_gather(c, buf, sem):
            return pltpu.async_copy(
                table_hbm.at[idx_v.at[pl.ds(c * chunk, chunk)]], buf, sem)

        gcp = [start_gather(0, bufs[0], gsems[0]), None]
        ocp = [None, None]
        for c in range(nch):
            cur = c & 1
            nxt = 1 - cur
            if c + 1 < nch:
                if ocp[nxt] is not None:
                    ocp[nxt].wait()
                gcp[nxt] = start_gather(c + 1, bufs[nxt], gsems[nxt])
            gcp[cur].wait()
            ocp[cur] = pltpu.async_copy(
                bufs[cur], out_hbm.at[pl.ds(base + c * chunk, chunk)],
                osems[cur])
        for cp in ocp:
            if cp is not None:
                cp.wait()

    return gather_kernel(table, idx)


def _tc_project_t(emb512, W_pad, bias_col, bblk):
    batch, fl = emb512.shape
    nrow = W_pad.shape[0]
    assert batch % bblk == 0

    def proj_kernel(emb_ref, w_ref, b_ref, out_ref):
        out_ref[...] = (
            lax.dot_general(w_ref[...], emb_ref[...],
                            (((1,), (1,)), ((), ())),
                            preferred_element_type=jnp.float32)
            + b_ref[...]
        )

    return pl.pallas_call(
        proj_kernel,
        grid=(batch // bblk,),
        in_specs=[
            pl.BlockSpec((bblk, fl), lambda i: (i, 0)),
            pl.BlockSpec((nrow, fl), lambda i: (0, 0)),
            pl.BlockSpec((nrow, 1), lambda i: (0, 0)),
        ],
        out_specs=pl.BlockSpec((nrow, bblk), lambda i: (0, i)),
        out_shape=jax.ShapeDtypeStruct((nrow, batch), jnp.float32),
        compiler_params=pltpu.CompilerParams(
            dimension_semantics=("parallel",)),
    )(emb512, W_pad, bias_col)


def kernel(x, A, B_w, B_b):
    batch, fields = x.shape
    vocab, rank = A.shape
    embed = B_w.shape[0]
    fpad = 32

    pack = LANES // rank
    table = _tc_pack_table(A.T, blkv=8192).reshape(vocab * pack, rank)

    idx = jnp.concatenate([x, x[:, :fpad - fields]], axis=1).reshape(
        batch * fpad) * pack
    emb = _sc_gather(table, idx, chunk=2048)
    emb512 = emb.reshape(batch, fpad * rank)

    W_pad = jnp.kron(jnp.eye(fields, fpad, dtype=jnp.float32),
                     B_w).astype(jnp.bfloat16)
    bias_col = jnp.tile(B_b, fields).reshape(fields * embed, 1)
    out_t = _tc_project_t(emb512, W_pad, bias_col, bblk=1024)

    return out_t.reshape(fields, embed, batch).transpose(2, 0, 1)


# speedup vs baseline: 4.6108x; 3.1855x over previous
import functools

import jax
import jax.numpy as jnp
from jax import lax
from jax.experimental import pallas as pl
from jax.experimental.pallas import tpu as pltpu
from jax.experimental.pallas import tpu_sc as plsc

NUM_CORES = 2
NUM_SUBCORES = 16
NW = NUM_CORES * NUM_SUBCORES
LANES = 128


def _tc_pack_table(At, blkv):
    d, vocab = At.shape
    assert blkv % LANES == 0
    grid = -(-vocab // blkv)

    def pack_kernel(at_ref, out_ref):
        out_ref[:, :d] = at_ref[...].T

    return pl.pallas_call(
        pack_kernel,
        grid=(grid,),
        in_specs=[pl.BlockSpec((d, blkv), lambda i: (0, i))],
        out_specs=pl.BlockSpec((blkv, LANES), lambda i: (i, 0)),
        out_shape=jax.ShapeDtypeStruct((vocab, LANES), jnp.float32),
        compiler_params=pltpu.CompilerParams(
            dimension_semantics=("parallel",)),
    )(At)


def _sc_gather(table, idx, chunk):
    n = idx.shape[0]
    d = table.shape[1]
    b_per_w = n // NW
    nch = b_per_w // chunk
    assert b_per_w * NW == n and nch * chunk == b_per_w and chunk % 8 == 0

    mesh = plsc.VectorSubcoreMesh(core_axis_name="c", subcore_axis_name="s")

    @functools.partial(
        pl.kernel,
        mesh=mesh,
        out_type=jax.ShapeDtypeStruct((n, d), table.dtype),
        scratch_types=[
            pltpu.VMEM((b_per_w,), jnp.int32),
            pltpu.VMEM((chunk, d), table.dtype),
            pltpu.VMEM((chunk, d), table.dtype),
            pltpu.SemaphoreType.DMA,
            pltpu.SemaphoreType.DMA,
            pltpu.SemaphoreType.DMA,
            pltpu.SemaphoreType.DMA,
        ],
        compiler_params=pltpu.CompilerParams(use_tc_tiling_on_sc=False),
    )
    def gather_kernel(table_hbm, idx_hbm, out_hbm, idx_v, buf0, buf1,
                      gs0, gs1, os0, os1):
        wid = lax.axis_index("s") * NUM_CORES + lax.axis_index("c")
        base = wid * b_per_w
        pltpu.sync_copy(idx_hbm.at[pl.ds(base, b_per_w)], idx_v)

        bufs = (buf0, buf1)
        gsems = (gs0, gs1)
        osems = (os0, os1)

        def start_gather(c, buf, sem):
            return pltpu.async_copy(
                table_hbm.at[idx_v.at[pl.ds(c * chunk, chunk)]], buf, sem)

        gcp = [start_gather(0, bufs[0], gsems[0]), None]
        ocp = [None, None]
        for c in range(nch):
            cur = c & 1
            nxt = 1 - cur
            if c + 1 < nch:
                if ocp[nxt] is not None:
                    ocp[nxt].wait()
                gcp[nxt] = start_gather(c + 1, bufs[nxt], gsems[nxt])
            gcp[cur].wait()
            ocp[cur] = pltpu.async_copy(
                bufs[cur], out_hbm.at[pl.ds(base + c * chunk, chunk)],
                osems[cur])
        for cp in ocp:
            if cp is not None:
                cp.wait()

    return gather_kernel(table, idx)


def _tc_project_t(emb512, W_pad, bias_col, bblk):
    batch, fl = emb512.shape
    nrow = W_pad.shape[0]
    assert batch % bblk == 0

    def proj_kernel(emb_ref, w_ref, b_ref, out_ref):
        out_ref[...] = (
            lax.dot_general(w_ref[...], emb_ref[...],
                            (((1,), (1,)), ((), ())),
                            preferred_element_type=jnp.float32)
            + b_ref[...]
        )

    return pl.pallas_call(
        proj_kernel,
        grid=(batch // bblk,),
        in_specs=[
            pl.BlockSpec((bblk, fl), lambda i: (i, 0)),
            pl.BlockSpec((nrow, fl), lambda i: (0, 0)),
            pl.BlockSpec((nrow, 1), lambda i: (0, 0)),
        ],
        out_specs=pl.BlockSpec((nrow, bblk), lambda i: (0, i)),
        out_shape=jax.ShapeDtypeStruct((nrow, batch), jnp.float32),
        compiler_params=pltpu.CompilerParams(
            dimension_semantics=("parallel",)),
    )(emb512, W_pad, bias_col)


def kernel(x, A, B_w, B_b):
    batch, fields = x.shape
    vocab, rank = A.shape
    embed = B_w.shape[0]
    fpad = 32

    pack = LANES // rank
    table = _tc_pack_table(A.T, blkv=8192).reshape(vocab * pack, rank)

    idx = jnp.concatenate([x, x[:, :fpad - fields]], axis=1).reshape(
        batch * fpad) * pack
    emb = _sc_gather(table, idx, chunk=2048)
    emb512 = emb.reshape(batch, fpad * rank)

    W_pad = jnp.kron(jnp.eye(fields, fpad, dtype=jnp.float32), B_w)
    bias_col = jnp.tile(B_b, fields).reshape(fields * embed, 1)
    out_t = _tc_project_t(emb512, W_pad, bias_col, bblk=1024)

    return out_t.reshape(fields, embed, batch).transpose(2, 0, 1)
